# Initial kernel scaffold; baseline (speedup 1.0000x reference)
import functools

import jax
import jax.numpy as jnp
from jax import lax
from jax.experimental import pallas as pl
from jax.experimental.pallas import tpu as pltpu
from jax.experimental.pallas import tpu_sc as plsc

EMB_D = 64
K = 1024
TPB = 1024
NB = 16
N_TOK = NB * TPB
COMMIT = 0.25


def _vq_tc_body(x_ref, wt_ref, xsq_ref, wsq_ref,
                idx_ref, hist_ref, counts_ref, loss_ref, perp_ref,
                acc_counts, acc_loss):
    b = pl.program_id(0)
    x = x_ref[...]
    wt = wt_ref[...]
    xsq = xsq_ref[...]
    wsq = wsq_ref[...]

    xw = jnp.dot(x, wt, preferred_element_type=jnp.float32)
    dist = (xsq + wsq) - 2.0 * xw

    minv = jnp.min(dist, axis=1, keepdims=True)
    lane = lax.broadcasted_iota(jnp.int32, (TPB, K), 1)
    idx = jnp.min(jnp.where(dist == minv, lane, K), axis=1, keepdims=True)
    idx_ref[...] = idx

    e = jnp.exp(minv - dist)
    s = jnp.sum(e, axis=1, keepdims=True)
    hist_ref[0] = jnp.sum(e / s, axis=0, keepdims=True)

    onehot = (lane == idx).astype(jnp.float32)
    counts_row = jnp.sum(onehot, axis=0, keepdims=True)
    counts_ref[0] = counts_row

    lp = jnp.sum(minv, axis=0, keepdims=True)

    @pl.when(b == 0)
    def _():
        acc_counts[...] = counts_row
        acc_loss[...] = lp
        loss_ref[...] = jnp.zeros((1, 1), jnp.float32)
        perp_ref[...] = jnp.zeros((1, 1), jnp.float32)

    @pl.when(b > 0)
    def _():
        acc_counts[...] += counts_row
        acc_loss[...] += lp

    @pl.when(b == NB - 1)
    def _():
        avg = acc_counts[...] * (1.0 / N_TOK)
        ent = jnp.sum(avg * jnp.log(avg + 1e-10), axis=1, keepdims=True)
        perp_ref[...] = jnp.exp(-ent)
        loss_ref[...] = acc_loss[...] * ((1.0 + COMMIT) / (N_TOK * EMB_D))


def _build_tc(interpret=False):
    return pl.pallas_call(
        _vq_tc_body,
        grid=(NB,),
        in_specs=[
            pl.BlockSpec((TPB, EMB_D), lambda b: (b, 0)),
            pl.BlockSpec((EMB_D, K), lambda b: (0, 0)),
            pl.BlockSpec((TPB, 1), lambda b: (b, 0)),
            pl.BlockSpec((1, K), lambda b: (0, 0)),
        ],
        out_specs=[
            pl.BlockSpec((TPB, 1), lambda b: (b, 0)),
            pl.BlockSpec((1, 1, K), lambda b: (b, 0, 0)),
            pl.BlockSpec((1, 1, K), lambda b: (b, 0, 0)),
            pl.BlockSpec((1, 1), lambda b: (0, 0)),
            pl.BlockSpec((1, 1), lambda b: (0, 0)),
        ],
        out_shape=[
            jax.ShapeDtypeStruct((N_TOK, 1), jnp.int32),
            jax.ShapeDtypeStruct((NB, 1, K), jnp.float32),
            jax.ShapeDtypeStruct((NB, 1, K), jnp.float32),
            jax.ShapeDtypeStruct((1, 1), jnp.float32),
            jax.ShapeDtypeStruct((1, 1), jnp.float32),
        ],
        scratch_shapes=[
            pltpu.VMEM((1, K), jnp.float32),
            pltpu.VMEM((1, 1), jnp.float32),
        ],
        interpret=interpret,
    )


def _make_sc_gather():
    info = plsc.get_sparse_core_info()
    nc, ns = info.num_cores, info.num_subcores
    nw = nc * ns
    b_per_w = N_TOK // nw
    mesh = plsc.VectorSubcoreMesh(core_axis_name="c", subcore_axis_name="s")

    @functools.partial(
        pl.kernel, mesh=mesh,
        out_type=jax.ShapeDtypeStruct((N_TOK, EMB_D), jnp.float32),
        scratch_types=[
            pltpu.VMEM((b_per_w,), jnp.int32),
            pltpu.VMEM((b_per_w, EMB_D), jnp.float32),
            pltpu.SemaphoreType.DMA,
        ],
    )
    def gather_rows(table_hbm, idx_hbm, out_hbm, idx_v, rows_v, sem):
        wid = lax.axis_index("s") * nc + lax.axis_index("c")
        base = wid * b_per_w
        pltpu.sync_copy(idx_hbm.at[pl.ds(base, b_per_w)], idx_v)
        pltpu.async_copy(table_hbm.at[idx_v], rows_v, sem).wait()
        pltpu.sync_copy(rows_v, out_hbm.at[pl.ds(base, b_per_w)])

    return gather_rows


def kernel(input, W):
    x = jnp.transpose(input, (0, 2, 3, 1)).reshape(N_TOK, EMB_D)
    wt = W.T
    xsq = jnp.sum(x ** 2, axis=1, keepdims=True)
    wsq = jnp.sum(W ** 2, axis=1)[None, :]

    idx, hist, counts, loss, perp = _build_tc()(x, wt, xsq, wsq)

    q = _make_sc_gather()(W, idx.reshape(N_TOK))
    quantized_out = jnp.transpose(q.reshape(NB, 32, 32, EMB_D), (0, 3, 1, 2))

    return (quantized_out, loss[0, 0], perp[0, 0], idx.reshape(NB, TPB),
            counts.reshape(NB, K), hist.reshape(NB, K))


# speedup vs baseline: 2.0474x; 2.0474x over previous
import functools

import jax
import jax.numpy as jnp
from jax import lax
from jax.experimental import pallas as pl
from jax.experimental.pallas import tpu as pltpu
from jax.experimental.pallas import tpu_sc as plsc

EMB_D = 64
K = 1024
TPB = 1024
NB = 16
N_TOK = NB * TPB
COMMIT = 0.25


def _vq_tc_body(x_ref, wt_ref, xsq_ref, wsq_ref,
                idx_ref, hist_ref, counts_ref, loss_ref, perp_ref,
                acc_counts, acc_loss):
    b = pl.program_id(0)
    x = x_ref[...]
    wt = wt_ref[...]
    xsq = xsq_ref[...]
    wsq = wsq_ref[...]

    xw = jnp.dot(x, wt, preferred_element_type=jnp.float32)
    dist = (xsq + wsq) - 2.0 * xw

    minv = jnp.min(dist, axis=1, keepdims=True)
    lane = lax.broadcasted_iota(jnp.int32, (TPB, K), 1)
    idx = jnp.min(jnp.where(dist == minv, lane, K), axis=1, keepdims=True)
    idx_ref[...] = idx

    e = jnp.exp(minv - dist)
    s = jnp.sum(e, axis=1, keepdims=True)
    hist_ref[0] = jnp.sum(e / s, axis=0, keepdims=True)

    onehot = (lane == idx).astype(jnp.float32)
    counts_row = jnp.sum(onehot, axis=0, keepdims=True)
    counts_ref[0] = counts_row

    lp = jnp.sum(minv, axis=0, keepdims=True)

    @pl.when(b == 0)
    def _():
        acc_counts[...] = counts_row
        acc_loss[...] = lp
        loss_ref[...] = jnp.zeros((1, 1), jnp.float32)
        perp_ref[...] = jnp.zeros((1, 1), jnp.float32)

    @pl.when(b > 0)
    def _():
        acc_counts[...] += counts_row
        acc_loss[...] += lp

    @pl.when(b == NB - 1)
    def _():
        avg = acc_counts[...] * (1.0 / N_TOK)
        ent = jnp.sum(avg * jnp.log(avg + 1e-10), axis=1, keepdims=True)
        perp_ref[...] = jnp.exp(-ent)
        loss_ref[...] = acc_loss[...] * ((1.0 + COMMIT) / (N_TOK * EMB_D))


def _build_tc(interpret=False):
    return pl.pallas_call(
        _vq_tc_body,
        grid=(NB,),
        in_specs=[
            pl.BlockSpec((TPB, EMB_D), lambda b: (b, 0)),
            pl.BlockSpec((EMB_D, K), lambda b: (0, 0)),
            pl.BlockSpec((TPB, 1), lambda b: (b, 0)),
            pl.BlockSpec((1, K), lambda b: (0, 0)),
        ],
        out_specs=[
            pl.BlockSpec((TPB, 1), lambda b: (b, 0)),
            pl.BlockSpec((1, 1, K), lambda b: (b, 0, 0)),
            pl.BlockSpec((1, 1, K), lambda b: (b, 0, 0)),
            pl.BlockSpec((1, 1), lambda b: (0, 0)),
            pl.BlockSpec((1, 1), lambda b: (0, 0)),
        ],
        out_shape=[
            jax.ShapeDtypeStruct((N_TOK, 1), jnp.int32),
            jax.ShapeDtypeStruct((NB, 1, K), jnp.float32),
            jax.ShapeDtypeStruct((NB, 1, K), jnp.float32),
            jax.ShapeDtypeStruct((1, 1), jnp.float32),
            jax.ShapeDtypeStruct((1, 1), jnp.float32),
        ],
        scratch_shapes=[
            pltpu.VMEM((1, K), jnp.float32),
            pltpu.VMEM((1, 1), jnp.float32),
        ],
        interpret=interpret,
    )


GATHER_D = 128


def _make_sc_gather():
    info = plsc.get_sparse_core_info()
    nc, ns = info.num_cores, info.num_subcores
    nw = nc * ns
    b_per_w = N_TOK // nw
    mesh = plsc.VectorSubcoreMesh(core_axis_name="c", subcore_axis_name="s")

    @functools.partial(
        pl.kernel, mesh=mesh,
        out_type=jax.ShapeDtypeStruct((N_TOK, GATHER_D), jnp.float32),
        scratch_types=[
            pltpu.VMEM((b_per_w,), jnp.int32),
            pltpu.VMEM((b_per_w, GATHER_D), jnp.float32),
            pltpu.SemaphoreType.DMA,
        ],
    )
    def gather_rows(table_hbm, idx_hbm, out_hbm, idx_v, rows_v, sem):
        wid = lax.axis_index("s") * nc + lax.axis_index("c")
        base = wid * b_per_w
        pltpu.sync_copy(idx_hbm.at[pl.ds(base, b_per_w)], idx_v)
        pltpu.async_copy(table_hbm.at[idx_v], rows_v, sem).wait()
        pltpu.sync_copy(rows_v, out_hbm.at[pl.ds(base, b_per_w)])

    return gather_rows


def kernel(input, W):
    x = jnp.transpose(input, (0, 2, 3, 1)).reshape(N_TOK, EMB_D)
    wt = W.T
    xsq = jnp.sum(x ** 2, axis=1, keepdims=True)
    wsq = jnp.sum(W ** 2, axis=1)[None, :]

    idx, hist, counts, loss, perp = _build_tc()(x, wt, xsq, wsq)

    w_pad = jnp.pad(W, ((0, 0), (0, GATHER_D - EMB_D)))
    q = _make_sc_gather()(w_pad, idx.reshape(N_TOK))[:, :EMB_D]
    quantized_out = jnp.transpose(q.reshape(NB, 32, 32, EMB_D), (0, 3, 1, 2))

    return (quantized_out, loss[0, 0], perp[0, 0], idx.reshape(NB, TPB),
            counts.reshape(NB, K), hist.reshape(NB, K))
